# baseline (speedup 1.0000x reference)
import functools

import jax
import jax.numpy as jnp
from jax import lax
from jax.experimental import pallas as pl
from jax.experimental.pallas import tpu as pltpu
from jax.experimental.pallas import tpu_sc as plsc

H = 128
CH = 128
NC = 2
NS = 16
NW = NC * NS
L = 16


NB = 4


def _lane_bcast(v, idx):
    return lax.gather(
        v, idx[:, None],
        lax.GatherDimensionNumbers(
            offset_dims=(), collapsed_slice_dims=(0,), start_index_map=(0,)),
        (1,), mode=lax.GatherScatterMode.PROMISE_IN_BOUNDS)


def _sc_centrality(xf, d0, d1, zcat):
    nodes = xf.shape[0]
    assert nodes % CH == 0
    nchunks = nodes // CH
    niter = (nchunks + NW - 1) // NW
    nslots = niter + 1
    nrounds = (nslots + NB - 1) // NB
    zrows = zcat.shape[0] // H

    mesh = plsc.VectorSubcoreMesh(core_axis_name="c", subcore_axis_name="s")

    @functools.partial(
        pl.kernel,
        out_type=jax.ShapeDtypeStruct((nodes, H), jnp.float32),
        mesh=mesh,
        compiler_params=pltpu.CompilerParams(needs_layout_passes=False),
        scratch_types=dict(
            zt=pltpu.VMEM((132 * 2 * H,), jnp.bfloat16),
            d0b=[pltpu.VMEM((CH,), jnp.int32)] * NB,
            d1b=[pltpu.VMEM((CH,), jnp.int32)] * NB,
            xb=[pltpu.VMEM((CH, H), jnp.float32)] * NB,
            sem_z=pltpu.SemaphoreType.DMA,
            sem_in=[pltpu.SemaphoreType.DMA] * NB,
            sem_o=[pltpu.SemaphoreType.DMA] * NB,
        ),
    )
    def k(x_hbm, d0_hbm, d1_hbm, z_hbm, out_hbm, *,
          zt, d0b, d1b, xb, sem_z, sem_in, sem_o):
        wid = lax.axis_index("s") * NC + lax.axis_index("c")

        cz = pltpu.async_copy(z_hbm, zt, sem_z)

        def in_copies(ic, p):
            base = (wid + ic * NW) * CH
            return (
                pltpu.make_async_copy(
                    d0_hbm.at[pl.ds(base, CH)], d0b[p], sem_in[p]),
                pltpu.make_async_copy(
                    d1_hbm.at[pl.ds(base, CH)], d1b[p], sem_in[p]),
                pltpu.make_async_copy(
                    x_hbm.at[pl.ds(base, CH), :], xb[p], sem_in[p]),
            )

        def o_copy(ic, p):
            base = (wid + ic * NW) * CH
            return pltpu.make_async_copy(
                xb[p], out_hbm.at[pl.ds(base, CH), :], sem_o[p])

        cz.wait()

        @pl.loop(0, nrounds)
        def rnd(r):
            for b in range(NB):
                i = r * NB + b

                @pl.when((i < niter) & (wid + i * NW < nchunks))
                def _():
                    @pl.when(i >= NB)
                    def _():
                        o_copy(i - NB, b).wait()
                    for c in in_copies(i, b):
                        c.start()

                ib, q = i - 1, (b - 1) % NB

                @pl.when((ib >= 0) & (ib < niter) & (wid + ib * NW < nchunks))
                def _():
                    for c in in_copies(ib, q):
                        c.wait()
                    lanes = lax.iota(jnp.int32, L)

                    @plsc.parallel_loop(0, CH // L, 1)
                    def grp(g):
                        gs = pl.ds(g * L, L)
                        d0v = d0b[q][gs]
                        d1v = d1b[q][gs]
                        pad = d0v == -1
                        iin_v = jnp.where(pad, 65, jnp.minimum(d1v, 64))
                        iout_v = jnp.where(
                            pad, zrows - 1, jnp.minimum(d0v, 64) + 66)
                        comb_v = iin_v * 65536 + iout_v
                        @plsc.parallel_loop(0, L, 1, unroll=4)
                        def lane(t):
                            sel = lanes == t
                            comb = jnp.max(jnp.where(sel, comb_v, 0))
                            i_in = comb >> 16
                            i_out = comb & 65535
                            n = g * L + t
                            for j2 in range(H // (2 * L)):
                                ai, bi = plsc.unpack(
                                    zt[pl.ds(i_in * (2 * H) + j2 * L, 2 * L)],
                                    format=plsc.PackFormat.INTERLEAVED)
                                ao, bo = plsc.unpack(
                                    zt[pl.ds(i_out * (2 * H) + j2 * L, 2 * L)],
                                    format=plsc.PackFormat.INTERLEAVED)
                                plsc.addupdate(
                                    xb[q].at[n, pl.ds(j2 * 2 * L, L)],
                                    ai + ao)
                                plsc.addupdate(
                                    xb[q].at[n, pl.ds(j2 * 2 * L + L, L)],
                                    bi + bo)

                    o_copy(ib, q).start()

        for t in range(NB):
            i = niter - NB + t

            @pl.when((i >= 0) & (wid + i * NW < nchunks))
            def _():
                o_copy(i, i % NB).wait()

    return k(xf, d0, d1, zcat)


def kernel(x, degrees, z_in, z_out):
    B, N, Hdim = x.shape
    zero = jnp.zeros((1, Hdim), jnp.float32)
    zcat = jnp.concatenate(
        [z_in.astype(jnp.float32), zero, z_out.astype(jnp.float32), zero], 0)
    zb = (zcat.reshape(zcat.shape[0], Hdim // 32, 2, 16)
          .transpose(0, 2, 1, 3).reshape(zcat.shape[0], 2, Hdim // 2))
    zcat = jnp.pad(zb, ((0, 0), (0, 0), (0, Hdim // 2))).astype(jnp.bfloat16)
    d0 = degrees[:, 0, :].reshape(-1).astype(jnp.int32)
    d1 = degrees[:, 1, :].reshape(-1).astype(jnp.int32)
    xf = x.reshape(-1, Hdim)
    out = _sc_centrality(xf, d0, d1, zcat.reshape(-1))
    return out.reshape(B, N, Hdim)


# speedup vs baseline: 1.0199x; 1.0199x over previous
import functools

import jax
import jax.numpy as jnp
from jax import lax
from jax.experimental import pallas as pl
from jax.experimental.pallas import tpu as pltpu
from jax.experimental.pallas import tpu_sc as plsc

H = 128
CH = 128
NC = 2
NS = 16
NW = NC * NS
L = 16


NB = 4


def _lane_bcast(v, idx):
    return lax.gather(
        v, idx[:, None],
        lax.GatherDimensionNumbers(
            offset_dims=(), collapsed_slice_dims=(0,), start_index_map=(0,)),
        (1,), mode=lax.GatherScatterMode.PROMISE_IN_BOUNDS)


def _sc_centrality(xf, dcomb, zcat):
    nodes = xf.shape[0]
    assert nodes % CH == 0
    nchunks = nodes // CH
    niter = (nchunks + NW - 1) // NW
    nslots = niter + 1
    nrounds = (nslots + NB - 1) // NB
    zrows = zcat.shape[0] // H

    mesh = plsc.VectorSubcoreMesh(core_axis_name="c", subcore_axis_name="s")

    @functools.partial(
        pl.kernel,
        out_type=jax.ShapeDtypeStruct((nodes, H), jnp.float32),
        mesh=mesh,
        compiler_params=pltpu.CompilerParams(needs_layout_passes=False),
        scratch_types=dict(
            zt=pltpu.VMEM((132 * 2 * H,), jnp.bfloat16),
            db=[pltpu.VMEM((CH,), jnp.int32)] * NB,
            xb=[pltpu.VMEM((CH, H), jnp.float32)] * NB,
            sem_z=pltpu.SemaphoreType.DMA,
            sem_in=[pltpu.SemaphoreType.DMA] * NB,
            sem_o=[pltpu.SemaphoreType.DMA] * NB,
        ),
    )
    def k(x_hbm, d_hbm, z_hbm, out_hbm, *,
          zt, db, xb, sem_z, sem_in, sem_o):
        wid = lax.axis_index("s") * NC + lax.axis_index("c")

        cz = pltpu.async_copy(z_hbm, zt, sem_z)

        def in_copies(ic, p):
            base = (wid + ic * NW) * CH
            return (
                pltpu.make_async_copy(
                    d_hbm.at[pl.ds(base, CH)], db[p], sem_in[p]),
                pltpu.make_async_copy(
                    x_hbm.at[pl.ds(base, CH), :], xb[p], sem_in[p]),
            )

        def o_copy(ic, p):
            base = (wid + ic * NW) * CH
            return pltpu.make_async_copy(
                xb[p], out_hbm.at[pl.ds(base, CH), :], sem_o[p])

        cz.wait()

        @pl.loop(0, nrounds)
        def rnd(r):
            for b in range(NB):
                i = r * NB + b

                @pl.when((i < niter) & (wid + i * NW < nchunks))
                def _():
                    @pl.when(i >= NB)
                    def _():
                        o_copy(i - NB, b).wait()
                    for c in in_copies(i, b):
                        c.start()

                ib, q = i - 1, (b - 1) % NB

                @pl.when((ib >= 0) & (ib < niter) & (wid + ib * NW < nchunks))
                def _():
                    for c in in_copies(ib, q):
                        c.wait()
                    lanes = lax.iota(jnp.int32, L)

                    @plsc.parallel_loop(0, CH // L, 1)
                    def grp(g):
                        gs = pl.ds(g * L, L)
                        cv = db[q][gs]
                        d0v = (cv >> 16) - 1
                        d1v = (cv & 65535) - 1
                        pad = d0v == -1
                        iin_v = jnp.where(pad, 65, jnp.minimum(d1v, 64))
                        iout_v = jnp.where(
                            pad, zrows - 1, jnp.minimum(d0v, 64) + 66)
                        comb_v = iin_v * 65536 + iout_v
                        @plsc.parallel_loop(0, L, 1, unroll=4)
                        def lane(t):
                            sel = lanes == t
                            comb = jnp.max(jnp.where(sel, comb_v, 0))
                            i_in = comb >> 16
                            i_out = comb & 65535
                            n = g * L + t
                            for j2 in range(H // (2 * L)):
                                ai, bi = plsc.unpack(
                                    zt[pl.ds(i_in * (2 * H) + j2 * L, 2 * L)],
                                    format=plsc.PackFormat.INTERLEAVED)
                                ao, bo = plsc.unpack(
                                    zt[pl.ds(i_out * (2 * H) + j2 * L, 2 * L)],
                                    format=plsc.PackFormat.INTERLEAVED)
                                plsc.addupdate(
                                    xb[q].at[n, pl.ds(j2 * 2 * L, L)],
                                    ai + ao)
                                plsc.addupdate(
                                    xb[q].at[n, pl.ds(j2 * 2 * L + L, L)],
                                    bi + bo)

                    o_copy(ib, q).start()

        for t in range(NB):
            i = niter - NB + t

            @pl.when((i >= 0) & (wid + i * NW < nchunks))
            def _():
                o_copy(i, i % NB).wait()

    return k(xf, dcomb, zcat)


def kernel(x, degrees, z_in, z_out):
    B, N, Hdim = x.shape
    zero = jnp.zeros((1, Hdim), jnp.float32)
    zcat = jnp.concatenate(
        [z_in.astype(jnp.float32), zero, z_out.astype(jnp.float32), zero], 0)
    zb = (zcat.reshape(zcat.shape[0], Hdim // 32, 2, 16)
          .transpose(0, 2, 1, 3).reshape(zcat.shape[0], 2, Hdim // 2))
    zcat = jnp.pad(zb, ((0, 0), (0, 0), (0, Hdim // 2))).astype(jnp.bfloat16)
    dg = degrees.astype(jnp.int32)
    dcomb = ((dg[:, 0, :] + 1) << 16) | ((dg[:, 1, :] + 1) & 0xFFFF)
    xf = x.reshape(-1, Hdim)
    out = _sc_centrality(xf, dcomb.reshape(-1), zcat.reshape(-1))
    return out.reshape(B, N, Hdim)


# speedup vs baseline: 1.0221x; 1.0022x over previous
import functools

import jax
import jax.numpy as jnp
from jax import lax
from jax.experimental import pallas as pl
from jax.experimental.pallas import tpu as pltpu
from jax.experimental.pallas import tpu_sc as plsc

H = 128
CH = 128
NC = 2
NS = 16
NW = NC * NS
L = 16


NB = 4


def _lane_bcast(v, idx):
    return lax.gather(
        v, idx[:, None],
        lax.GatherDimensionNumbers(
            offset_dims=(), collapsed_slice_dims=(0,), start_index_map=(0,)),
        (1,), mode=lax.GatherScatterMode.PROMISE_IN_BOUNDS)


def _sc_centrality(xf, dcomb, zcat):
    nodes = xf.shape[0]
    assert nodes % CH == 0
    nchunks = nodes // CH
    niter = (nchunks + NW - 1) // NW
    nslots = niter + 1
    nrounds = (nslots + NB - 1) // NB
    zrows = zcat.shape[0] // H

    mesh = plsc.VectorSubcoreMesh(core_axis_name="c", subcore_axis_name="s")

    @functools.partial(
        pl.kernel,
        out_type=jax.ShapeDtypeStruct((nodes, H), jnp.float32),
        mesh=mesh,
        compiler_params=pltpu.CompilerParams(
            needs_layout_passes=False, skip_device_barrier=True),
        scratch_types=dict(
            zt=pltpu.VMEM((132 * 2 * H,), jnp.bfloat16),
            db=[pltpu.VMEM((CH,), jnp.int32)] * NB,
            xb=[pltpu.VMEM((CH, H), jnp.float32)] * NB,
            sem_z=pltpu.SemaphoreType.DMA,
            sem_in=[pltpu.SemaphoreType.DMA] * NB,
            sem_o=[pltpu.SemaphoreType.DMA] * NB,
        ),
    )
    def k(x_hbm, d_hbm, z_hbm, out_hbm, *,
          zt, db, xb, sem_z, sem_in, sem_o):
        wid = lax.axis_index("s") * NC + lax.axis_index("c")

        cz = pltpu.async_copy(z_hbm, zt, sem_z)

        def in_copies(ic, p):
            base = (wid + ic * NW) * CH
            return (
                pltpu.make_async_copy(
                    d_hbm.at[pl.ds(base, CH)], db[p], sem_in[p]),
                pltpu.make_async_copy(
                    x_hbm.at[pl.ds(base, CH), :], xb[p], sem_in[p]),
            )

        def o_copy(ic, p):
            base = (wid + ic * NW) * CH
            return pltpu.make_async_copy(
                xb[p], out_hbm.at[pl.ds(base, CH), :], sem_o[p])

        cz.wait()

        @pl.loop(0, nrounds)
        def rnd(r):
            for b in range(NB):
                i = r * NB + b

                @pl.when((i < niter) & (wid + i * NW < nchunks))
                def _():
                    @pl.when(i >= NB)
                    def _():
                        o_copy(i - NB, b).wait()
                    for c in in_copies(i, b):
                        c.start()

                ib, q = i - 1, (b - 1) % NB

                @pl.when((ib >= 0) & (ib < niter) & (wid + ib * NW < nchunks))
                def _():
                    for c in in_copies(ib, q):
                        c.wait()
                    lanes = lax.iota(jnp.int32, L)

                    @plsc.parallel_loop(0, CH // L, 1)
                    def grp(g):
                        gs = pl.ds(g * L, L)
                        cv = db[q][gs]
                        d0v = (cv >> 16) - 1
                        d1v = (cv & 65535) - 1
                        pad = d0v == -1
                        iin_v = jnp.where(pad, 65, jnp.minimum(d1v, 64))
                        iout_v = jnp.where(
                            pad, zrows - 1, jnp.minimum(d0v, 64) + 66)
                        comb_v = iin_v * 65536 + iout_v
                        @plsc.parallel_loop(0, L, 1, unroll=4)
                        def lane(t):
                            sel = lanes == t
                            comb = jnp.max(jnp.where(sel, comb_v, 0))
                            i_in = comb >> 16
                            i_out = comb & 65535
                            n = g * L + t
                            for j2 in range(H // (2 * L)):
                                ai, bi = plsc.unpack(
                                    zt[pl.ds(i_in * (2 * H) + j2 * L, 2 * L)],
                                    format=plsc.PackFormat.INTERLEAVED)
                                ao, bo = plsc.unpack(
                                    zt[pl.ds(i_out * (2 * H) + j2 * L, 2 * L)],
                                    format=plsc.PackFormat.INTERLEAVED)
                                plsc.addupdate(
                                    xb[q].at[n, pl.ds(j2 * 2 * L, L)],
                                    ai + ao)
                                plsc.addupdate(
                                    xb[q].at[n, pl.ds(j2 * 2 * L + L, L)],
                                    bi + bo)

                    o_copy(ib, q).start()

        for t in range(NB):
            i = niter - NB + t

            @pl.when((i >= 0) & (wid + i * NW < nchunks))
            def _():
                o_copy(i, i % NB).wait()

    return k(xf, dcomb, zcat)


def kernel(x, degrees, z_in, z_out):
    B, N, Hdim = x.shape
    zero = jnp.zeros((1, Hdim), jnp.float32)
    zcat = jnp.concatenate(
        [z_in.astype(jnp.float32), zero, z_out.astype(jnp.float32), zero], 0)
    zb = (zcat.reshape(zcat.shape[0], Hdim // 32, 2, 16)
          .transpose(0, 2, 1, 3).reshape(zcat.shape[0], 2, Hdim // 2))
    zcat = jnp.pad(zb, ((0, 0), (0, 0), (0, Hdim // 2))).astype(jnp.bfloat16)
    dg = degrees.astype(jnp.int32)
    dcomb = ((dg[:, 0, :] + 1) << 16) | ((dg[:, 1, :] + 1) & 0xFFFF)
    xf = x.reshape(-1, Hdim)
    out = _sc_centrality(xf, dcomb.reshape(-1), zcat.reshape(-1))
    return out.reshape(B, N, Hdim)


# speedup vs baseline: 1.0499x; 1.0271x over previous
import functools

import jax
import jax.numpy as jnp
from jax import lax
from jax.experimental import pallas as pl
from jax.experimental.pallas import tpu as pltpu
from jax.experimental.pallas import tpu_sc as plsc

H = 128
CH = 128
NC = 2
NS = 16
NW = NC * NS
L = 16


NB = 4


def _lane_bcast(v, idx):
    return lax.gather(
        v, idx[:, None],
        lax.GatherDimensionNumbers(
            offset_dims=(), collapsed_slice_dims=(0,), start_index_map=(0,)),
        (1,), mode=lax.GatherScatterMode.PROMISE_IN_BOUNDS)


def _sc_centrality(xf, dcomb, zi, zo):
    nodes = xf.shape[0]
    assert nodes % CH == 0
    nchunks = nodes // CH
    niter = (nchunks + NW - 1) // NW
    nslots = niter + 1
    nrounds = (nslots + NB - 1) // NB
    zrows = 132

    mesh = plsc.VectorSubcoreMesh(core_axis_name="c", subcore_axis_name="s")

    @functools.partial(
        pl.kernel,
        out_type=jax.ShapeDtypeStruct((nodes, H), jnp.float32),
        mesh=mesh,
        compiler_params=pltpu.CompilerParams(
            needs_layout_passes=False, skip_device_barrier=True),
        scratch_types=dict(
            zt=pltpu.VMEM((132 * 2 * H,), jnp.bfloat16),
            zfi=pltpu.VMEM((65 * H,), jnp.float32),
            zfo=pltpu.VMEM((65 * H,), jnp.float32),
            db=[pltpu.VMEM((CH,), jnp.int32)] * NB,
            xb=[pltpu.VMEM((CH, H), jnp.float32)] * NB,
            sem_z=pltpu.SemaphoreType.DMA,
            sem_in=[pltpu.SemaphoreType.DMA] * NB,
            sem_o=[pltpu.SemaphoreType.DMA] * NB,
        ),
    )
    def k(x_hbm, d_hbm, zi_hbm, zo_hbm, out_hbm, *,
          zt, zfi, zfo, db, xb, sem_z, sem_in, sem_o):
        wid = lax.axis_index("s") * NC + lax.axis_index("c")

        cza = pltpu.async_copy(zi_hbm, zfi, sem_z)
        czb = pltpu.async_copy(zo_hbm, zfo, sem_z)

        def in_copies(ic, p):
            base = (wid + ic * NW) * CH
            return (
                pltpu.make_async_copy(
                    d_hbm.at[pl.ds(base, CH)], db[p], sem_in[p]),
                pltpu.make_async_copy(
                    x_hbm.at[pl.ds(base, CH), :], xb[p], sem_in[p]),
            )

        def o_copy(ic, p):
            base = (wid + ic * NW) * CH
            return pltpu.make_async_copy(
                xb[p], out_hbm.at[pl.ds(base, CH), :], sem_o[p])

        @pl.loop(0, nrounds)
        def rnd(r):
            for b in range(NB):
                i = r * NB + b

                @pl.when((i < niter) & (wid + i * NW < nchunks))
                def _():
                    @pl.when(i >= NB)
                    def _():
                        o_copy(i - NB, b).wait()
                    for c in in_copies(i, b):
                        c.start()

                ib, q = i - 1, (b - 1) % NB

                @pl.when((ib >= 0) & (ib < niter) & (wid + ib * NW < nchunks))
                def _():
                    @pl.when(ib == 0)
                    def _():
                        cza.wait()
                        czb.wait()
                        zero2l = jnp.zeros((2 * L,), jnp.bfloat16)

                        @plsc.parallel_loop(0, 65, 1)
                        def bld(r):
                            for j in range(H // (2 * L)):
                                src = r * H + 2 * L * j
                                zt[pl.ds(r * 2 * H + L * j, 2 * L)] = (
                                    plsc.pack(
                                        zfi[pl.ds(src, L)],
                                        zfi[pl.ds(src + L, L)],
                                        format=plsc.PackFormat.INTERLEAVED))
                                zt[pl.ds((r + 66) * 2 * H + L * j, 2 * L)] = (
                                    plsc.pack(
                                        zfo[pl.ds(src, L)],
                                        zfo[pl.ds(src + L, L)],
                                        format=plsc.PackFormat.INTERLEAVED))

                        for j in range(H // (2 * L)):
                            zt[pl.ds(65 * 2 * H + L * j, 2 * L)] = zero2l
                            zt[pl.ds(131 * 2 * H + L * j, 2 * L)] = zero2l

                    for c in in_copies(ib, q):
                        c.wait()
                    lanes = lax.iota(jnp.int32, L)

                    @plsc.parallel_loop(0, CH // L, 1)
                    def grp(g):
                        gs = pl.ds(g * L, L)
                        cv = db[q][gs]
                        d0v = (cv >> 16) - 1
                        d1v = (cv & 65535) - 1
                        pad = d0v == -1
                        iin_v = jnp.where(pad, 65, jnp.minimum(d1v, 64))
                        iout_v = jnp.where(
                            pad, zrows - 1, jnp.minimum(d0v, 64) + 66)
                        comb_v = iin_v * 65536 + iout_v
                        @plsc.parallel_loop(0, L, 1, unroll=4)
                        def lane(t):
                            sel = lanes == t
                            comb = jnp.max(jnp.where(sel, comb_v, 0))
                            i_in = comb >> 16
                            i_out = comb & 65535
                            n = g * L + t
                            for j2 in range(H // (2 * L)):
                                ai, bi = plsc.unpack(
                                    zt[pl.ds(i_in * (2 * H) + j2 * L, 2 * L)],
                                    format=plsc.PackFormat.INTERLEAVED)
                                ao, bo = plsc.unpack(
                                    zt[pl.ds(i_out * (2 * H) + j2 * L, 2 * L)],
                                    format=plsc.PackFormat.INTERLEAVED)
                                plsc.addupdate(
                                    xb[q].at[n, pl.ds(j2 * 2 * L, L)],
                                    ai + ao)
                                plsc.addupdate(
                                    xb[q].at[n, pl.ds(j2 * 2 * L + L, L)],
                                    bi + bo)

                    o_copy(ib, q).start()

        for t in range(NB):
            i = niter - NB + t

            @pl.when((i >= 0) & (wid + i * NW < nchunks))
            def _():
                o_copy(i, i % NB).wait()

    return k(xf, dcomb, zi, zo)


def kernel(x, degrees, z_in, z_out):
    B, N, Hdim = x.shape
    dg = degrees.astype(jnp.int32)
    dcomb = ((dg[:, 0, :] + 1) << 16) | ((dg[:, 1, :] + 1) & 0xFFFF)
    xf = x.reshape(-1, Hdim)
    out = _sc_centrality(xf, dcomb.reshape(-1),
                         z_in.astype(jnp.float32).reshape(-1),
                         z_out.astype(jnp.float32).reshape(-1))
    return out.reshape(B, N, Hdim)


# speedup vs baseline: 1.0510x; 1.0010x over previous
import functools

import jax
import jax.numpy as jnp
from jax import lax
from jax.experimental import pallas as pl
from jax.experimental.pallas import tpu as pltpu
from jax.experimental.pallas import tpu_sc as plsc

H = 128
CH = 128
NC = 2
NS = 16
NW = NC * NS
L = 16


NB = 4


def _lane_bcast(v, idx):
    return lax.gather(
        v, idx[:, None],
        lax.GatherDimensionNumbers(
            offset_dims=(), collapsed_slice_dims=(0,), start_index_map=(0,)),
        (1,), mode=lax.GatherScatterMode.PROMISE_IN_BOUNDS)


def _sc_centrality(xf, dcomb, zi, zo):
    nodes = xf.shape[0]
    assert nodes % CH == 0
    nchunks = nodes // CH
    niter = (nchunks + NW - 1) // NW
    nslots = niter + 1
    nrounds = (nslots + NB - 1) // NB
    zrows = 132

    mesh = plsc.VectorSubcoreMesh(core_axis_name="c", subcore_axis_name="s")

    @functools.partial(
        pl.kernel,
        out_type=jax.ShapeDtypeStruct((nodes, H), jnp.float32),
        mesh=mesh,
        compiler_params=pltpu.CompilerParams(
            needs_layout_passes=False, skip_device_barrier=True),
        scratch_types=dict(
            zt=pltpu.VMEM((132 * 2 * H,), jnp.bfloat16),
            zfi=pltpu.VMEM((65 * H,), jnp.float32),
            zfo=pltpu.VMEM((65 * H,), jnp.float32),
            db=[pltpu.VMEM((CH,), jnp.int32)] * NB,
            xb=[pltpu.VMEM((CH, H), jnp.float32)] * NB,
            sem_z=pltpu.SemaphoreType.DMA,
            sem_in=[pltpu.SemaphoreType.DMA] * NB,
            sem_o=[pltpu.SemaphoreType.DMA] * NB,
        ),
    )
    def k(x_hbm, d_hbm, zi_hbm, zo_hbm, out_hbm, *,
          zt, zfi, zfo, db, xb, sem_z, sem_in, sem_o):
        wid = lax.axis_index("s") * NC + lax.axis_index("c")

        cza = pltpu.async_copy(zi_hbm, zfi, sem_z)
        czb = pltpu.async_copy(zo_hbm, zfo, sem_z)

        def in_copies(ic, p):
            base = (wid + ic * NW) * CH
            return (
                pltpu.make_async_copy(
                    d_hbm.at[pl.ds(base, CH)], db[p], sem_in[p]),
                pltpu.make_async_copy(
                    x_hbm.at[pl.ds(base, CH), :], xb[p], sem_in[p]),
            )

        def o_copy(ic, p):
            base = (wid + ic * NW) * CH
            return pltpu.make_async_copy(
                xb[p], out_hbm.at[pl.ds(base, CH), :], sem_o[p])

        @pl.loop(0, nrounds)
        def rnd(r):
            for b in range(NB):
                i = r * NB + b

                @pl.when((i < niter) & (wid + i * NW < nchunks))
                def _():
                    @pl.when(i >= NB)
                    def _():
                        o_copy(i - NB, b).wait()
                    for c in in_copies(i, b):
                        c.start()

                ib, q = i - 1, (b - 1) % NB

                @pl.when((ib >= 0) & (ib < niter) & (wid + ib * NW < nchunks))
                def _():
                    @pl.when(ib == 0)
                    def _():
                        cza.wait()
                        czb.wait()
                        zero2l = jnp.zeros((2 * L,), jnp.bfloat16)

                        @plsc.parallel_loop(0, 65, 1)
                        def bld(r):
                            for j in range(H // (2 * L)):
                                src = r * H + 2 * L * j
                                zt[pl.ds(r * 2 * H + L * j, 2 * L)] = (
                                    plsc.pack(
                                        zfi[pl.ds(src, L)],
                                        zfi[pl.ds(src + L, L)],
                                        format=plsc.PackFormat.INTERLEAVED))
                                zt[pl.ds((r + 66) * 2 * H + L * j, 2 * L)] = (
                                    plsc.pack(
                                        zfo[pl.ds(src, L)],
                                        zfo[pl.ds(src + L, L)],
                                        format=plsc.PackFormat.INTERLEAVED))

                        for j in range(H // (2 * L)):
                            zt[pl.ds(65 * 2 * H + L * j, 2 * L)] = zero2l
                            zt[pl.ds(131 * 2 * H + L * j, 2 * L)] = zero2l

                    for c in in_copies(ib, q):
                        c.wait()
                    lanes = lax.iota(jnp.int32, L)

                    @plsc.parallel_loop(0, CH // L, 1)
                    def grp(g):
                        gs = pl.ds(g * L, L)
                        cv = db[q][gs]
                        d0v = (cv >> 16) - 1
                        d1v = (cv & 65535) - 1
                        pad = d0v == -1
                        iin_v = jnp.where(pad, 65, jnp.minimum(d1v, 64))
                        iout_v = jnp.where(
                            pad, zrows - 1, jnp.minimum(d0v, 64) + 66)
                        comb_v = iin_v * 65536 + iout_v
                        @plsc.parallel_loop(0, L, 1, unroll=4)
                        def lane(t):
                            sel = lanes == t
                            comb = jnp.max(jnp.where(sel, comb_v, 0))
                            i_in = comb >> 16
                            i_out = comb & 65535
                            n = g * L + t
                            for j2 in range(H // (2 * L)):
                                ai, bi = plsc.unpack(
                                    zt[pl.ds(i_in * (2 * H) + j2 * L, 2 * L)],
                                    format=plsc.PackFormat.INTERLEAVED)
                                ao, bo = plsc.unpack(
                                    zt[pl.ds(i_out * (2 * H) + j2 * L, 2 * L)],
                                    format=plsc.PackFormat.INTERLEAVED)
                                plsc.addupdate(
                                    xb[q].at[n, pl.ds(j2 * 2 * L, L)],
                                    ai + ao)
                                plsc.addupdate(
                                    xb[q].at[n, pl.ds(j2 * 2 * L + L, L)],
                                    bi + bo)

                    o_copy(ib, q).start()

        for t in range(NB):
            i = niter - NB + t

            @pl.when((i >= 0) & (wid + i * NW < nchunks))
            def _():
                o_copy(i, i % NB).wait()

    return k(xf, dcomb, zi, zo)


def kernel(x, degrees, z_in, z_out):
    B, N, Hdim = x.shape
    dg = degrees.astype(jnp.int32)
    d0f = dg[:, 0, :].reshape(-1)
    d1f = dg[:, 1, :].reshape(-1)
    dcomb = ((d0f + 1) << 16) | ((d1f + 1) & 0xFFFF)
    xf = x.reshape(-1, Hdim)
    out = _sc_centrality(xf, dcomb,
                         z_in.astype(jnp.float32).reshape(-1),
                         z_out.astype(jnp.float32).reshape(-1))
    return out.reshape(B, N, Hdim)
